# baseline (speedup 1.0000x reference)
import jax
import jax.numpy as jnp
from jax import lax
from jax.experimental import pallas as pl
from jax.experimental.pallas import tpu as pltpu
from jax.experimental.pallas import tpu_sc as plsc

B = 16384
L = 20
D = 32
NC, NS, LANES = 2, 16, 16
NW = NC * NS
BPW = B // NW
NTC = 128
RB = 32
CPB = RB * L // NTC
NB = BPW // RB
NG = BPW // LANES


def _body(ids_hbm, tok_hbm, ttab_hbm, ktab_hbm, out_hbm,
          ids_v, tok_v, e1_v, out_v, gat_v, p_v, q_v, t0_v, sems, gsem):
    wid = lax.axis_index("s") * NC + lax.axis_index("c")
    base = wid * BPW

    pltpu.sync_copy(ids_hbm.at[pl.ds(base, BPW)], ids_v)
    pltpu.sync_copy(tok_hbm.at[pl.ds(base * L, BPW * L)], tok_v)
    pltpu.sync_copy(ktab_hbm.at[pl.ds(0, 8)], t0_v)

    e1_copies = [
        pltpu.async_copy(
            ttab_hbm.at[ids_v.at[pl.ds(j * NTC, NTC)]],
            e1_v.at[pl.ds(j * NTC, NTC)], gsem)
        for j in range(BPW // NTC)
    ]

    def fire(jb, buf):
        t0 = jb * RB * L
        return [
            pltpu.async_copy(
                ktab_hbm.at[tok_v.at[pl.ds(t0 + c * NTC, NTC)]],
                gat_v.at[buf, pl.ds(c * NTC, NTC)], sems.at[buf])
            for c in range(CPB)
        ]

    inflight = [fire(0, 0), fire(1, 1)]

    lane20 = lax.iota(jnp.int32, LANES) * L

    @pl.loop(0, NG)
    def _(g):
        s = g * LANES
        idx0 = s * L + lane20
        cnt = jnp.zeros((LANES,), jnp.float32)
        for l in range(L):
            t = plsc.load_gather(tok_v, [idx0 + l])
            cnt = cnt + (t != 0).astype(jnp.float32)
        inv = 1.0 / jnp.maximum(cnt, 1.0)
        p_v[pl.ds(s, LANES)] = inv
        q_v[pl.ds(s, LANES)] = (float(L) - cnt) * inv

    for c in e1_copies:
        c.wait()

    for jb in range(NB):
        buf = jb % 3
        for c in inflight[jb]:
            c.wait()
        if jb + 2 < NB:
            inflight.append(fire(jb + 2, (jb + 2) % 3))
        else:
            inflight.append([])

        @pl.loop(0, RB)
        def _(rr, jb=jb, buf=buf):
            r = jb * RB + rr
            rowb = rr * L
            a0 = gat_v[buf, rowb, pl.ds(0, LANES)]
            a1 = gat_v[buf, rowb, pl.ds(LANES, LANES)]
            for l in range(1, L):
                a0 = a0 + gat_v[buf, rowb + l, pl.ds(0, LANES)]
                a1 = a1 + gat_v[buf, rowb + l, pl.ds(LANES, LANES)]
            rb = r * (2 * D)
            out_v[pl.ds(rb, LANES)] = e1_v[r, pl.ds(0, LANES)]
            out_v[pl.ds(rb + LANES, LANES)] = e1_v[r, pl.ds(LANES, LANES)]
            ridx = jnp.full((LANES,), r, jnp.int32)
            p = plsc.load_gather(p_v, [ridx])
            q = plsc.load_gather(q_v, [ridx])
            t0a = t0_v[0, pl.ds(0, LANES)]
            t0b = t0_v[0, pl.ds(LANES, LANES)]
            out_v[pl.ds(rb + D, LANES)] = a0 * p - q * t0a
            out_v[pl.ds(rb + D + LANES, LANES)] = a1 * p - q * t0b

    pltpu.sync_copy(out_v, out_hbm.at[pl.ds(base * 2 * D, BPW * 2 * D)])


@jax.jit
def kernel(title_ids, title_tokens, title_table, token_table):
    tokens_flat = title_tokens.reshape(-1).astype(jnp.int32)
    mesh = plsc.VectorSubcoreMesh(core_axis_name="c", subcore_axis_name="s")
    k = pl.kernel(
        _body,
        out_type=jax.ShapeDtypeStruct((B * 2 * D,), jnp.float32),
        mesh=mesh,
        compiler_params=pltpu.CompilerParams(
            use_tc_tiling_on_sc=False, needs_layout_passes=False),
        scratch_types=[
            pltpu.VMEM((BPW,), jnp.int32),
            pltpu.VMEM((BPW * L,), jnp.int32),
            pltpu.VMEM((BPW, D), jnp.float32),
            pltpu.VMEM((BPW * 2 * D,), jnp.float32),
            pltpu.VMEM((3, RB * L, D), jnp.float32),
            pltpu.VMEM((BPW,), jnp.float32),
            pltpu.VMEM((BPW,), jnp.float32),
            pltpu.VMEM((8, D), jnp.float32),
            pltpu.SemaphoreType.DMA((3,)),
            pltpu.SemaphoreType.DMA,
        ],
    )
    out = k(title_ids.astype(jnp.int32), tokens_flat, title_table, token_table)
    return out.reshape(B, 2 * D)


# speedup vs baseline: 1.0623x; 1.0623x over previous
import jax
import jax.numpy as jnp
from jax import lax
from jax.experimental import pallas as pl
from jax.experimental.pallas import tpu as pltpu
from jax.experimental.pallas import tpu_sc as plsc

B = 16384
L = 20
D = 32
NC, NS, LANES = 2, 16, 16
NW = NC * NS
BPW = B // NW
RB = 32
NB = BPW // RB
NG = BPW // LANES
NTC = 128


def _wid_base():
    wid = lax.axis_index("s") * NC + lax.axis_index("c")
    return wid * BPW


def _tok_body(tok_hbm, ktab_hbm, e2_hbm,
              tok_v, gat_v, ev, p_v, q_v, t0_v, sems):
    base = _wid_base()

    pltpu.sync_copy(tok_hbm.at[pl.ds(base * L, BPW * L)], tok_v)
    pltpu.sync_copy(ktab_hbm.at[pl.ds(0, 8)], t0_v)

    def fire(jb, buf):
        t0 = jb * RB * L
        return [
            pltpu.async_copy(
                ktab_hbm.at[tok_v.at[pl.ds(t0 + c * NTC, NTC)]],
                gat_v.at[buf, pl.ds(c * NTC, NTC)], sems.at[buf])
            for c in range(RB * L // NTC)
        ]

    inflight = [fire(0, 0), fire(1, 1)]

    lane20 = lax.iota(jnp.int32, LANES) * L

    @pl.loop(0, NG)
    def _(g):
        s = g * LANES
        idx0 = s * L + lane20
        cnt = jnp.zeros((LANES,), jnp.float32)
        for l in range(L):
            t = plsc.load_gather(tok_v, [idx0 + l])
            cnt = cnt + (t != 0).astype(jnp.float32)
        inv = 1.0 / jnp.maximum(cnt, 1.0)
        p_v[pl.ds(s, LANES)] = inv
        q_v[pl.ds(s, LANES)] = (float(L) - cnt) * inv

    for jb in range(NB):
        buf = jb % 3
        for c in inflight[jb]:
            c.wait()
        if jb + 2 < NB:
            inflight.append(fire(jb + 2, (jb + 2) % 3))
        else:
            inflight.append([])

        @pl.loop(0, RB)
        def _(rr, jb=jb, buf=buf):
            r = jb * RB + rr
            rowb = rr * L
            a0 = gat_v[buf, rowb, pl.ds(0, LANES)]
            a1 = gat_v[buf, rowb, pl.ds(LANES, LANES)]
            for l in range(1, L):
                a0 = a0 + gat_v[buf, rowb + l, pl.ds(0, LANES)]
                a1 = a1 + gat_v[buf, rowb + l, pl.ds(LANES, LANES)]
            ridx = jnp.full((LANES,), r, jnp.int32)
            p = plsc.load_gather(p_v, [ridx])
            q = plsc.load_gather(q_v, [ridx])
            t0a = t0_v[0, pl.ds(0, LANES)]
            t0b = t0_v[0, pl.ds(LANES, LANES)]
            rb = r * D
            ev[pl.ds(rb, LANES)] = a0 * p - q * t0a
            ev[pl.ds(rb + LANES, LANES)] = a1 * p - q * t0b

    pltpu.sync_copy(ev, e2_hbm.at[pl.ds(base * D, BPW * D)])


def _title_body(ids_hbm, ttab_hbm, e2_hbm, out_hbm,
                ids_v, e1_v, e2_v, out_v, gsem, esem):
    base = _wid_base()

    pltpu.sync_copy(ids_hbm.at[pl.ds(base, BPW)], ids_v)
    copies = [
        pltpu.async_copy(
            ttab_hbm.at[ids_v.at[pl.ds(j * NTC, NTC)]],
            e1_v.at[pl.ds(j * NTC, NTC)], gsem)
        for j in range(BPW // NTC)
    ]
    e2c = pltpu.async_copy(e2_hbm.at[pl.ds(base * D, BPW * D)], e2_v, esem)
    for c in copies:
        c.wait()
    e2c.wait()

    @pl.loop(0, BPW)
    def _(r):
        rb = r * (2 * D)
        rd = r * D
        out_v[pl.ds(rb, LANES)] = e1_v[r, pl.ds(0, LANES)]
        out_v[pl.ds(rb + LANES, LANES)] = e1_v[r, pl.ds(LANES, LANES)]
        out_v[pl.ds(rb + D, LANES)] = e2_v[pl.ds(rd, LANES)]
        out_v[pl.ds(rb + D + LANES, LANES)] = e2_v[pl.ds(rd + LANES, LANES)]

    pltpu.sync_copy(out_v, out_hbm.at[pl.ds(base * 2 * D, BPW * 2 * D)])


_MESH = plsc.VectorSubcoreMesh(core_axis_name="c", subcore_axis_name="s")
_CP = pltpu.CompilerParams(use_tc_tiling_on_sc=False, needs_layout_passes=False)


@jax.jit
def kernel(title_ids, title_tokens, title_table, token_table):
    k1 = pl.kernel(
        _tok_body,
        out_type=jax.ShapeDtypeStruct((B * D,), jnp.float32),
        mesh=_MESH,
        compiler_params=_CP,
        scratch_types=[
            pltpu.VMEM((BPW * L,), jnp.int32),
            pltpu.VMEM((3, RB * L, D), jnp.float32),
            pltpu.VMEM((BPW * D,), jnp.float32),
            pltpu.VMEM((BPW,), jnp.float32),
            pltpu.VMEM((BPW,), jnp.float32),
            pltpu.VMEM((8, D), jnp.float32),
            pltpu.SemaphoreType.DMA((3,)),
        ],
    )
    k2 = pl.kernel(
        _title_body,
        out_type=jax.ShapeDtypeStruct((B * 2 * D,), jnp.float32),
        mesh=_MESH,
        compiler_params=_CP,
        scratch_types=[
            pltpu.VMEM((BPW,), jnp.int32),
            pltpu.VMEM((BPW, D), jnp.float32),
            pltpu.VMEM((BPW * D,), jnp.float32),
            pltpu.VMEM((BPW * 2 * D,), jnp.float32),
            pltpu.SemaphoreType.DMA,
            pltpu.SemaphoreType.DMA,
        ],
    )
    e2 = k1(title_tokens.reshape(-1).astype(jnp.int32), token_table)
    out = k2(title_ids.astype(jnp.int32), title_table, e2)
    return out.reshape(B, 2 * D)


# speedup vs baseline: 1.0626x; 1.0002x over previous
import jax
import jax.numpy as jnp
from jax import lax
from jax.experimental import pallas as pl
from jax.experimental.pallas import tpu as pltpu
from jax.experimental.pallas import tpu_sc as plsc

B = 16384
L = 20
D = 32
NC, NS, LANES = 2, 16, 16
NW = NC * NS
BPW = B // NW
RB = 32
NB = BPW // RB
NG = BPW // LANES
NTC = 128


def _wid_base():
    wid = lax.axis_index("s") * NC + lax.axis_index("c")
    return wid * BPW


def _tok_body(tok_hbm, ktab_hbm, e2_hbm,
              tok_v, gat_v, ev, p_v, q_v, t0_v, sems):
    base = _wid_base()

    pltpu.sync_copy(tok_hbm.at[pl.ds(base * L, BPW * L)], tok_v)
    pltpu.sync_copy(ktab_hbm.at[pl.ds(0, 8)], t0_v)

    def fire(jb, buf):
        t0 = jb * RB * L
        return [
            pltpu.async_copy(
                ktab_hbm.at[tok_v.at[pl.ds(t0 + c * NTC, NTC)]],
                gat_v.at[buf, pl.ds(c * NTC, NTC)], sems.at[buf])
            for c in range(RB * L // NTC)
        ]

    inflight = [fire(0, 0), fire(1, 1)]

    lane20 = lax.iota(jnp.int32, LANES) * L

    @pl.loop(0, NG)
    def _(g):
        s = g * LANES
        idx0 = s * L + lane20
        cnt = jnp.zeros((LANES,), jnp.float32)
        for l in range(L):
            t = plsc.load_gather(tok_v, [idx0 + l])
            cnt = cnt + (t != 0).astype(jnp.float32)
        inv = 1.0 / jnp.maximum(cnt, 1.0)
        p_v[pl.ds(s, LANES)] = inv
        q_v[pl.ds(s, LANES)] = (float(L) - cnt) * inv

    for jb in range(NB):
        buf = jb % 3
        for c in inflight[jb]:
            c.wait()
        if jb + 2 < NB:
            inflight.append(fire(jb + 2, (jb + 2) % 3))
        else:
            inflight.append([])

        @pl.loop(0, RB)
        def _(rr, jb=jb, buf=buf):
            r = jb * RB + rr
            rowb = rr * L
            a0 = gat_v[buf, rowb, pl.ds(0, LANES)]
            a1 = gat_v[buf, rowb, pl.ds(LANES, LANES)]
            for l in range(1, L):
                a0 = a0 + gat_v[buf, rowb + l, pl.ds(0, LANES)]
                a1 = a1 + gat_v[buf, rowb + l, pl.ds(LANES, LANES)]
            ridx = jnp.full((LANES,), r, jnp.int32)
            p = plsc.load_gather(p_v, [ridx])
            q = plsc.load_gather(q_v, [ridx])
            t0a = t0_v[0, pl.ds(0, LANES)]
            t0b = t0_v[0, pl.ds(LANES, LANES)]
            rb = r * D
            ev[pl.ds(rb, LANES)] = a0 * p - q * t0a
            ev[pl.ds(rb + LANES, LANES)] = a1 * p - q * t0b

    pltpu.sync_copy(ev, e2_hbm.at[pl.ds(base * D, BPW * D)])


def _title_body(ids_hbm, ttab_hbm, e2_hbm, out_hbm,
                ids_v, e1_v, e2_v, out_v, gsem, esem):
    base = _wid_base()

    pltpu.sync_copy(ids_hbm.at[pl.ds(base, BPW)], ids_v)
    copies = [
        pltpu.async_copy(
            ttab_hbm.at[ids_v.at[pl.ds(j * NTC, NTC)]],
            e1_v.at[pl.ds(j * NTC, NTC)], gsem)
        for j in range(BPW // NTC)
    ]
    e2c = pltpu.async_copy(e2_hbm.at[pl.ds(base * D, BPW * D)], e2_v, esem)
    for c in copies:
        c.wait()
    e2c.wait()

    @pl.loop(0, BPW)
    def _(r):
        rd = r * D
        out_v[r, pl.ds(0, LANES)] = e1_v[r, pl.ds(0, LANES)]
        out_v[r, pl.ds(LANES, LANES)] = e1_v[r, pl.ds(LANES, LANES)]
        out_v[r, pl.ds(D, LANES)] = e2_v[pl.ds(rd, LANES)]
        out_v[r, pl.ds(D + LANES, LANES)] = e2_v[pl.ds(rd + LANES, LANES)]

    pltpu.sync_copy(out_v, out_hbm.at[pl.ds(base, BPW)])


_MESH = plsc.VectorSubcoreMesh(core_axis_name="c", subcore_axis_name="s")
_CP = pltpu.CompilerParams(use_tc_tiling_on_sc=False, needs_layout_passes=False)


@jax.jit
def kernel(title_ids, title_tokens, title_table, token_table):
    k1 = pl.kernel(
        _tok_body,
        out_type=jax.ShapeDtypeStruct((B * D,), jnp.float32),
        mesh=_MESH,
        compiler_params=_CP,
        scratch_types=[
            pltpu.VMEM((BPW * L,), jnp.int32),
            pltpu.VMEM((3, RB * L, D), jnp.float32),
            pltpu.VMEM((BPW * D,), jnp.float32),
            pltpu.VMEM((BPW,), jnp.float32),
            pltpu.VMEM((BPW,), jnp.float32),
            pltpu.VMEM((8, D), jnp.float32),
            pltpu.SemaphoreType.DMA((3,)),
        ],
    )
    k2 = pl.kernel(
        _title_body,
        out_type=jax.ShapeDtypeStruct((B, 2 * D), jnp.float32),
        mesh=_MESH,
        compiler_params=_CP,
        scratch_types=[
            pltpu.VMEM((BPW,), jnp.int32),
            pltpu.VMEM((BPW, D), jnp.float32),
            pltpu.VMEM((BPW * D,), jnp.float32),
            pltpu.VMEM((BPW, 2 * D), jnp.float32),
            pltpu.SemaphoreType.DMA,
            pltpu.SemaphoreType.DMA,
        ],
    )
    e2 = k1(title_tokens.reshape(-1).astype(jnp.int32), token_table)
    return k2(title_ids.astype(jnp.int32), title_table, e2)


# speedup vs baseline: 1.1236x; 1.0574x over previous
import jax
import jax.numpy as jnp
from jax import lax
from jax.experimental import pallas as pl
from jax.experimental.pallas import tpu as pltpu
from jax.experimental.pallas import tpu_sc as plsc

B = 16384
L = 20
D = 32
NC, NS, LANES = 2, 16, 16
NW = NC * NS
BPW = B // NW
RB = 32
NB = BPW // RB
NG = BPW // LANES
NTC = 128


def _wid_base():
    wid = lax.axis_index("s") * NC + lax.axis_index("c")
    return wid * BPW


def _tok_body(tok_hbm, ktab_hbm, e2_hbm,
              tok_v, gat_v, ev, p_v, q_v, t0_v, ssem, sems):
    base = _wid_base()

    stage = [
        pltpu.async_copy(tok_hbm.at[pl.ds(l, 1), pl.ds(base, BPW)],
                         tok_v.at[pl.ds(l, 1)], ssem)
        for l in range(L)
    ]
    pltpu.sync_copy(ktab_hbm.at[pl.ds(0, 8)], t0_v)
    for c in stage:
        c.wait()

    def fire(jb, buf):
        s = jb * RB
        return [
            pltpu.async_copy(
                ktab_hbm.at[tok_v.at[l, pl.ds(s, RB)]],
                gat_v.at[buf, l], sems.at[buf])
            for l in range(L)
        ]

    inflight = [fire(0, 0), fire(1, 1)]

    @pl.loop(0, NG)
    def _(g):
        s = g * LANES
        cnt = jnp.zeros((LANES,), jnp.float32)
        for l in range(L):
            t = tok_v[l, pl.ds(s, LANES)]
            cnt = cnt + (t != 0).astype(jnp.float32)
        inv = 1.0 / jnp.maximum(cnt, 1.0)
        p_v[pl.ds(s, LANES)] = inv
        q_v[pl.ds(s, LANES)] = (float(L) - cnt) * inv

    for jb in range(NB):
        buf = jb % 3
        for c in inflight[jb]:
            c.wait()
        if jb + 2 < NB:
            inflight.append(fire(jb + 2, (jb + 2) % 3))
        else:
            inflight.append([])

        @pl.loop(0, RB)
        def _(rr, jb=jb, buf=buf):
            r = jb * RB + rr
            a0 = gat_v[buf, 0, rr, pl.ds(0, LANES)]
            a1 = gat_v[buf, 0, rr, pl.ds(LANES, LANES)]
            for l in range(1, L):
                a0 = a0 + gat_v[buf, l, rr, pl.ds(0, LANES)]
                a1 = a1 + gat_v[buf, l, rr, pl.ds(LANES, LANES)]
            ridx = jnp.full((LANES,), r, jnp.int32)
            p = plsc.load_gather(p_v, [ridx])
            q = plsc.load_gather(q_v, [ridx])
            t0a = t0_v[0, pl.ds(0, LANES)]
            t0b = t0_v[0, pl.ds(LANES, LANES)]
            rb = r * D
            ev[pl.ds(rb, LANES)] = a0 * p - q * t0a
            ev[pl.ds(rb + LANES, LANES)] = a1 * p - q * t0b

    pltpu.sync_copy(ev, e2_hbm.at[pl.ds(base * D, BPW * D)])


def _title_body(ids_hbm, ttab_hbm, e2_hbm, out_hbm,
                ids_v, e1_v, e2_v, out_v, gsem, esem):
    base = _wid_base()

    pltpu.sync_copy(ids_hbm.at[pl.ds(base, BPW)], ids_v)
    copies = [
        pltpu.async_copy(
            ttab_hbm.at[ids_v.at[pl.ds(j * NTC, NTC)]],
            e1_v.at[pl.ds(j * NTC, NTC)], gsem)
        for j in range(BPW // NTC)
    ]
    e2c = pltpu.async_copy(e2_hbm.at[pl.ds(base * D, BPW * D)], e2_v, esem)
    for c in copies:
        c.wait()
    e2c.wait()

    @pl.loop(0, BPW)
    def _(r):
        rd = r * D
        out_v[r, pl.ds(0, LANES)] = e1_v[r, pl.ds(0, LANES)]
        out_v[r, pl.ds(LANES, LANES)] = e1_v[r, pl.ds(LANES, LANES)]
        out_v[r, pl.ds(D, LANES)] = e2_v[pl.ds(rd, LANES)]
        out_v[r, pl.ds(D + LANES, LANES)] = e2_v[pl.ds(rd + LANES, LANES)]

    pltpu.sync_copy(out_v, out_hbm.at[pl.ds(base, BPW)])


_MESH = plsc.VectorSubcoreMesh(core_axis_name="c", subcore_axis_name="s")
_CP = pltpu.CompilerParams(use_tc_tiling_on_sc=False, needs_layout_passes=False)


@jax.jit
def kernel(title_ids, title_tokens, title_table, token_table):
    k1 = pl.kernel(
        _tok_body,
        out_type=jax.ShapeDtypeStruct((B * D,), jnp.float32),
        mesh=_MESH,
        compiler_params=_CP,
        scratch_types=[
            pltpu.VMEM((L, BPW), jnp.int32),
            pltpu.VMEM((3, L, RB, D), jnp.float32),
            pltpu.VMEM((BPW * D,), jnp.float32),
            pltpu.VMEM((BPW,), jnp.float32),
            pltpu.VMEM((BPW,), jnp.float32),
            pltpu.VMEM((8, D), jnp.float32),
            pltpu.SemaphoreType.DMA,
            pltpu.SemaphoreType.DMA((3,)),
        ],
    )
    k2 = pl.kernel(
        _title_body,
        out_type=jax.ShapeDtypeStruct((B, 2 * D), jnp.float32),
        mesh=_MESH,
        compiler_params=_CP,
        scratch_types=[
            pltpu.VMEM((BPW,), jnp.int32),
            pltpu.VMEM((BPW, D), jnp.float32),
            pltpu.VMEM((BPW * D,), jnp.float32),
            pltpu.VMEM((BPW, 2 * D), jnp.float32),
            pltpu.SemaphoreType.DMA,
            pltpu.SemaphoreType.DMA,
        ],
    )
    e2 = k1(title_tokens.T.astype(jnp.int32), token_table)
    return k2(title_ids.astype(jnp.int32), title_table, e2)
